# baseline (speedup 1.0000x reference)
import jax
import jax.numpy as jnp
from jax import lax
from jax.experimental import pallas as pl
from jax.experimental.pallas import tpu as pltpu
from jax.experimental.pallas import tpu_sc as plsc

_FEAT = 64
_SEQ = 200
_BATCH = 4096
_BCOLS = _BATCH // 32


def _make_kernel():
    mesh = plsc.VectorSubcoreMesh(core_axis_name="c", subcore_axis_name="s")

    def body(tok_hbm, tab_hbm, out_hbm, idx_v, rows_v, pane_v, *sems):
        gsem = sems[0:2]
        wsem = sems[2:4]
        wid = lax.axis_index("s") * 2 + lax.axis_index("c")
        b0 = wid * _BCOLS
        pltpu.sync_copy(tok_hbm.at[:, pl.ds(b0, _BCOLS)], idx_v)

        lane = lax.iota(jnp.int32, 16)

        def gather(t, b):
            return pltpu.make_async_copy(
                tab_hbm.at[idx_v.at[t]], rows_v.at[b], gsem[b])

        def write(t, b):
            return pltpu.make_async_copy(
                pane_v.at[b],
                out_hbm.at[pl.ds(t, 1), :, pl.ds(b0, _BCOLS)],
                wsem[b])

        for b in range(2):
            gather(b, b).start()

        def step(t, carry):
            for b in range(2):
                tt = t * 2 + b
                gather(tt, b).wait()

                for f in range(_FEAT):
                    for k in range(_BCOLS // 16):
                        src = plsc.load_gather(
                            rows_v.at[b], [k * 16 + lane, lane * 0 + f])
                        pane_v[b, 0, f, pl.ds(k * 16, 16)] = src

                @pl.when(tt + 2 < _SEQ)
                def _():
                    gather(tt + 2, b).start()

                @pl.when(tt >= 2)
                def _():
                    write(tt - 2, b).wait()

                write(tt, b).start()
            return carry

        lax.fori_loop(0, _SEQ // 2, step, 0)
        for b in range(2):
            write(_SEQ - 2 + b, b).wait()

    return pl.kernel(
        body,
        out_type=jax.ShapeDtypeStruct((_SEQ, _FEAT, _BATCH), jnp.float32),
        mesh=mesh,
        compiler_params=pltpu.CompilerParams(
            use_tc_tiling_on_sc=False, needs_layout_passes=False),
        scratch_types=(
            [
                pltpu.VMEM((_SEQ, _BCOLS), jnp.int32),
                pltpu.VMEM((2, _BCOLS, _FEAT), jnp.float32),
                pltpu.VMEM((2, 1, _FEAT, _BCOLS), jnp.float32),
            ]
            + [pltpu.SemaphoreType.DMA] * 4
        ),
    )


def kernel(token_ids, emb_matrix):
    tok_t = token_ids.T.astype(jnp.int32)
    out3 = _make_kernel()(tok_t, emb_matrix)
    return out3.transpose(2, 0, 1)


# speedup vs baseline: 1.4905x; 1.4905x over previous
import jax
import jax.numpy as jnp
from jax import lax
from jax.experimental import pallas as pl
from jax.experimental.pallas import tpu as pltpu
from jax.experimental.pallas import tpu_sc as plsc

_FEAT = 64
_SEQ = 200
_BATCH = 4096
_BCOLS = _BATCH // 32


def _make_kernel():
    mesh = plsc.VectorSubcoreMesh(core_axis_name="c", subcore_axis_name="s")

    def body(tok_hbm, tab_hbm, out_hbm, idx_v, rows_v, pane_v, *sems):
        gsem = sems[0:2]
        wsem = sems[2:4]
        wid = lax.axis_index("s") * 2 + lax.axis_index("c")
        b0 = wid * _BCOLS
        pltpu.sync_copy(tok_hbm.at[:, pl.ds(b0, _BCOLS)], idx_v)

        lane = lax.iota(jnp.int32, 16)

        def gather(t, b):
            return pltpu.make_async_copy(
                tab_hbm.at[idx_v.at[t]], rows_v.at[b], gsem[b])

        def write(t, b):
            return pltpu.make_async_copy(
                pane_v.at[b],
                out_hbm.at[pl.ds(t, 1), :, pl.ds(b0, _BCOLS)],
                wsem[b])

        for b in range(2):
            gather(b, b).start()

        def step(t, carry):
            for b in range(2):
                tt = t * 2 + b
                gather(tt, b).wait()

                @plsc.parallel_loop(0, _FEAT, unroll=8)
                def _(f):
                    for k in range(_BCOLS // 16):
                        src = plsc.load_gather(
                            rows_v.at[b], [k * 16 + lane, lane * 0 + f])
                        pane_v[b, 0, f, pl.ds(k * 16, 16)] = src

                @pl.when(tt + 2 < _SEQ)
                def _():
                    gather(tt + 2, b).start()

                @pl.when(tt >= 2)
                def _():
                    write(tt - 2, b).wait()

                write(tt, b).start()
            return carry

        lax.fori_loop(0, _SEQ // 2, step, 0)
        for b in range(2):
            write(_SEQ - 2 + b, b).wait()

    return pl.kernel(
        body,
        out_type=jax.ShapeDtypeStruct((_SEQ, _FEAT, _BATCH), jnp.float32),
        mesh=mesh,
        compiler_params=pltpu.CompilerParams(
            use_tc_tiling_on_sc=False, needs_layout_passes=False),
        scratch_types=(
            [
                pltpu.VMEM((_SEQ, _BCOLS), jnp.int32),
                pltpu.VMEM((2, _BCOLS, _FEAT), jnp.float32),
                pltpu.VMEM((2, 1, _FEAT, _BCOLS), jnp.float32),
            ]
            + [pltpu.SemaphoreType.DMA] * 4
        ),
    )


def kernel(token_ids, emb_matrix):
    tok_t = token_ids.T.astype(jnp.int32)
    out3 = _make_kernel()(tok_t, emb_matrix)
    return out3.transpose(2, 0, 1)


# speedup vs baseline: 1.4954x; 1.0033x over previous
import jax
import jax.numpy as jnp
from jax import lax
from jax.experimental import pallas as pl
from jax.experimental.pallas import tpu as pltpu
from jax.experimental.pallas import tpu_sc as plsc

_FEAT = 64
_SEQ = 200
_BATCH = 4096
_BCOLS = _BATCH // 32
_NBUF = 4


def _make_kernel():
    mesh = plsc.VectorSubcoreMesh(core_axis_name="c", subcore_axis_name="s")

    def body(tok_hbm, tab_hbm, out_hbm, idx_v, rows_v, pane_v, *sems):
        gsem = sems[0:_NBUF]
        wsem = sems[_NBUF:]
        wid = lax.axis_index("s") * 2 + lax.axis_index("c")
        b0 = wid * _BCOLS
        pltpu.sync_copy(tok_hbm.at[:, pl.ds(b0, _BCOLS)], idx_v)

        lane = lax.iota(jnp.int32, 16)

        def gather(t, b):
            return pltpu.make_async_copy(
                tab_hbm.at[idx_v.at[t]], rows_v.at[b], gsem[b])

        def write(t, b):
            return pltpu.make_async_copy(
                pane_v.at[b],
                out_hbm.at[pl.ds(t, 1), :, pl.ds(b0, _BCOLS)],
                wsem[b])

        for b in range(_NBUF):
            gather(b, b).start()

        def step(i, carry):
            for b in range(_NBUF):
                tt = i * _NBUF + b
                gather(tt, b).wait()

                @pl.when(tt >= _NBUF)
                def _():
                    write(tt - _NBUF, b).wait()

                @plsc.parallel_loop(0, _FEAT, unroll=16)
                def _(f):
                    for k in range(_BCOLS // 16):
                        src = plsc.load_gather(
                            rows_v.at[b], [k * 16 + lane, lane * 0 + f])
                        pane_v[b, 0, f, pl.ds(k * 16, 16)] = src

                write(tt, b).start()

                @pl.when(tt + _NBUF < _SEQ)
                def _():
                    gather(tt + _NBUF, b).start()
            return carry

        lax.fori_loop(0, _SEQ // _NBUF, step, 0)
        for b in range(_NBUF):
            write(_SEQ - _NBUF + b, b).wait()

    return pl.kernel(
        body,
        out_type=jax.ShapeDtypeStruct((_SEQ, _FEAT, _BATCH), jnp.float32),
        mesh=mesh,
        compiler_params=pltpu.CompilerParams(
            use_tc_tiling_on_sc=False, needs_layout_passes=False),
        scratch_types=(
            [
                pltpu.VMEM((_SEQ, _BCOLS), jnp.int32),
                pltpu.VMEM((_NBUF, _BCOLS, _FEAT), jnp.float32),
                pltpu.VMEM((_NBUF, 1, _FEAT, _BCOLS), jnp.float32),
            ]
            + [pltpu.SemaphoreType.DMA] * (2 * _NBUF)
        ),
    )


def kernel(token_ids, emb_matrix):
    tok_t = token_ids.T.astype(jnp.int32)
    out3 = _make_kernel()(tok_t, emb_matrix)
    return out3.transpose(2, 0, 1)


# speedup vs baseline: 1.8149x; 1.2136x over previous
import jax
import jax.numpy as jnp
from jax import lax
from jax.experimental import pallas as pl
from jax.experimental.pallas import tpu as pltpu
from jax.experimental.pallas import tpu_sc as plsc

_EMB_DIM = 64
_SEQ = 200
_NSEQ = 4096
_SPW = _NSEQ // 32
_NBUF = 4


def _make_gather():
    mesh = plsc.VectorSubcoreMesh(core_axis_name="c", subcore_axis_name="s")
    bpw = _SPW * _SEQ

    def body(idx_hbm, table_hbm, out_hbm, idx_v, rows_v, *sems):
        gsem = sems[:_NBUF]
        wsem = sems[_NBUF:]
        wid = lax.axis_index("s") * 2 + lax.axis_index("c")
        s0 = wid * _SPW
        pltpu.sync_copy(idx_hbm.at[pl.ds(s0 * _SEQ, bpw)], idx_v)

        def gather(j, b):
            idx_slice = idx_v.at[pl.ds(j * _SEQ, _SEQ)]
            return pltpu.make_async_copy(
                table_hbm.at[idx_slice], rows_v.at[b], gsem[b])

        def write(j, b):
            return pltpu.make_async_copy(
                rows_v.at[b], out_hbm.at[s0 + j], wsem[b])

        for b in range(_NBUF):
            gather(b, b).start()

        def outer(i, carry):
            for k in range(_NBUF):
                j = i * _NBUF + k
                gather(j, k).wait()

                @pl.when(j >= _NBUF)
                def _():
                    write(j - _NBUF, k).wait()

                write(j, k).start()

                @pl.when(j + _NBUF < _SPW)
                def _():
                    gather(j + _NBUF, k).start()
            return carry

        lax.fori_loop(0, _SPW // _NBUF, outer, 0)
        for k in range(_NBUF):
            write(_SPW - _NBUF + k, (_SPW - _NBUF + k) % _NBUF).wait()

    return pl.kernel(
        body,
        out_type=jax.ShapeDtypeStruct((_NSEQ, _SEQ, _EMB_DIM), jnp.float32),
        mesh=mesh,
        compiler_params=pltpu.CompilerParams(use_tc_tiling_on_sc=False),
        scratch_types=(
            [
                pltpu.VMEM((bpw,), jnp.int32),
                pltpu.VMEM((_NBUF, _SEQ, _EMB_DIM), jnp.float32),
            ]
            + [pltpu.SemaphoreType.DMA] * (2 * _NBUF)
        ),
    )


def kernel(token_ids, emb_matrix):
    flat = token_ids.reshape(_NSEQ * _SEQ).astype(jnp.int32)
    return _make_gather()(flat, emb_matrix)
